# Initial kernel scaffold; baseline (speedup 1.0000x reference)
import jax
import jax.numpy as jnp
from jax.experimental import pallas as pl


def kernel(feature, logits, targets, feature_table, logit_table, count):
    raise NotImplementedError("write your pallas kernel here")


# speedup vs baseline: 1010.3792x; 1010.3792x over previous
import jax
import jax.numpy as jnp
from jax import lax
from jax.experimental import pallas as pl

NCLS = 1000
DIM = 512
BATCH = 4096
EPS = 1e-8
BB = 512
NB = BATCH // BB


def _stage1(f_ref, l_ref, t_ref, fs_ref, fh_ref, ls_ref, n_ref, e_ref):
    @pl.when(pl.program_id(0) == 0)
    def _init():
        fs_ref[...] = jnp.zeros_like(fs_ref)
        fh_ref[...] = jnp.zeros_like(fh_ref)
        ls_ref[...] = jnp.zeros_like(ls_ref)
        n_ref[...] = jnp.zeros_like(n_ref)
        e_ref[...] = jnp.zeros_like(e_ref)

    f = f_ref[...]
    l = l_ref[...]
    t = t_ref[0]
    oh = (lax.broadcasted_iota(jnp.int32, (NCLS, BB), 0) == t).astype(jnp.float32)

    nrm = jnp.sqrt(jnp.sum(f * f, axis=1, keepdims=True))
    fhat = f / jnp.maximum(nrm, EPS)
    m = jnp.max(l, axis=1, keepdims=True)
    lse = m + jnp.log(jnp.sum(jnp.exp(l - m), axis=1, keepdims=True))

    dn = (((1,), (0,)), ((), ()))
    fs_ref[...] += lax.dot_general(oh, f, dn, preferred_element_type=jnp.float32)
    fh_ref[...] += lax.dot_general(oh, fhat, dn, preferred_element_type=jnp.float32)
    ls_ref[...] += lax.dot_general(oh, l, dn, preferred_element_type=jnp.float32)
    n_ref[...] += jnp.sum(oh, axis=1, keepdims=True)
    e_ref[...] += lax.dot_general(oh, lse, dn, preferred_element_type=jnp.float32)


def _stage2(fs_ref, fh_ref, ls_ref, n_ref, e_ref, ftab_ref, ltab_ref, cnt_ref,
            l1_ref, l2_ref, ft_ref, lt_ref):
    fs = fs_ref[...]
    fh = fh_ref[...]
    ls = ls_ref[...]
    n = n_ref[...]
    lsesum = e_ref[...]
    ftab = ftab_ref[...]
    ltab = ltab_ref[...]
    cnt0 = cnt_ref[...]

    pos = n > 0.0
    denom = jnp.where(pos, cnt0 + n, 1.0)
    ft = jnp.where(pos, (ftab * cnt0 + fs) / denom, ftab)
    lt = jnp.where(pos, (ltab * cnt0 + ls) / denom, ltab)
    ft_ref[...] = ft
    lt_ref[...] = lt

    nft = jnp.sqrt(jnp.sum(ft * ft, axis=1, keepdims=True))
    dots = jnp.sum(ft * fh, axis=1, keepdims=True)
    fcl = float(BATCH) - jnp.sum(dots / jnp.maximum(nft, EPS))

    dnT = (((1,), (1,)), ((), ()))
    p = lax.dot_general(ft, ft, dnT, preferred_element_type=jnp.float32)
    outer = lax.dot_general(nft, nft, dnT, preferred_element_type=jnp.float32)
    cos = p / outer
    mn = jnp.min(cos)
    mx = jnp.max(cos)
    ct = (cos - mn) / (mx - mn)
    row = lax.broadcasted_iota(jnp.int32, (NCLS, NCLS), 0)
    col = lax.broadcasted_iota(jnp.int32, (NCLS, NCLS), 1)
    ct = jnp.where(row == col, 0.0, ct)

    scv = jnp.max(ct, axis=1, keepdims=True)
    sc_idx = jnp.min(jnp.where(ct == scv, col, NCLS), axis=1, keepdims=True)
    first = (col == sc_idx).astype(jnp.float32)

    fsc = lax.dot_general(first, ft, (((1,), (0,)), ((), ())),
                          preferred_element_type=jnp.float32)
    nsc = lax.dot_general(first, nft, (((1,), (0,)), ((), ())),
                          preferred_element_type=jnp.float32)
    sum1 = jnp.sum(jnp.sum(fsc * fh, axis=1, keepdims=True)
                   / jnp.maximum(nsc, EPS))
    sum2 = jnp.sum(n * scv)
    l1_ref[...] = jnp.broadcast_to(fcl + sum1 * sum2, (1, 128))

    lm = jnp.max(lt, axis=1, keepdims=True)
    ex = jnp.exp(lt - lm)
    q = ex / jnp.sum(ex, axis=1, keepdims=True)
    logq = jnp.log(q)
    negent = jnp.sum(q * logq, axis=1, keepdims=True)
    term1 = jnp.sum(n * negent)
    term2 = jnp.sum(q * (ls - lsesum))
    l2_ref[...] = jnp.broadcast_to(term1 - term2, (1, 128))


def kernel(feature, logits, targets, feature_table, logit_table, count):
    targets3 = targets.reshape(NB, 1, BB)
    f32 = jnp.float32
    fs, fh, ls, n, e = pl.pallas_call(
        _stage1,
        grid=(NB,),
        in_specs=[
            pl.BlockSpec((BB, DIM), lambda i: (i, 0)),
            pl.BlockSpec((BB, NCLS), lambda i: (i, 0)),
            pl.BlockSpec((1, 1, BB), lambda i: (i, 0, 0)),
        ],
        out_specs=[
            pl.BlockSpec((NCLS, DIM), lambda i: (0, 0)),
            pl.BlockSpec((NCLS, DIM), lambda i: (0, 0)),
            pl.BlockSpec((NCLS, NCLS), lambda i: (0, 0)),
            pl.BlockSpec((NCLS, 1), lambda i: (0, 0)),
            pl.BlockSpec((NCLS, 1), lambda i: (0, 0)),
        ],
        out_shape=[
            jax.ShapeDtypeStruct((NCLS, DIM), f32),
            jax.ShapeDtypeStruct((NCLS, DIM), f32),
            jax.ShapeDtypeStruct((NCLS, NCLS), f32),
            jax.ShapeDtypeStruct((NCLS, 1), f32),
            jax.ShapeDtypeStruct((NCLS, 1), f32),
        ],
    )(feature, logits, targets3)

    l1, l2, ft, lt = pl.pallas_call(
        _stage2,
        out_shape=[
            jax.ShapeDtypeStruct((1, 128), f32),
            jax.ShapeDtypeStruct((1, 128), f32),
            jax.ShapeDtypeStruct((NCLS, DIM), f32),
            jax.ShapeDtypeStruct((NCLS, NCLS), f32),
        ],
    )(fs, fh, ls, n, e, feature_table, logit_table, count)

    return (l1[0, 0], l2[0, 0], ft, lt)
